# baseline (speedup 1.0000x reference)
import functools

import jax
import jax.numpy as jnp
from jax import lax
from jax.experimental import pallas as pl
from jax.experimental.pallas import tpu as pltpu
from jax.experimental.pallas import tpu_sc as plsc

D_MODEL = 128
NC, NS = 2, 16
NW = NC * NS
CHUNK = 64
K = 14


def _make_lookup(B):
    assert B % (NW * CHUNK) == 0
    n_chunks = B // (NW * CHUNK)
    n_groups, rem = divmod(n_chunks - K, K)
    mesh = plsc.VectorSubcoreMesh(core_axis_name="c", subcore_axis_name="s")

    @functools.partial(
        pl.kernel,
        out_type=jax.ShapeDtypeStruct((B, D_MODEL), jnp.float32),
        mesh=mesh,
        scratch_types=[
            pltpu.VMEM((n_chunks, CHUNK), jnp.int32),
            pltpu.VMEM((K, CHUNK, D_MODEL), jnp.float32),
        ]
        + [pltpu.SemaphoreType.DMA] * K,
    )
    def lookup(idx_hbm, table_hbm, out_hbm, idx_v, bufs, *gsems):
        wid = lax.axis_index("s") * NC + lax.axis_index("c")
        base = wid * (n_chunks * CHUNK)
        pltpu.sync_copy(idx_hbm.at[wid], idx_v)

        def gather_fire(j, b):
            pltpu.async_copy(table_hbm.at[idx_v.at[j]], bufs.at[b], gsems[b])

        def gather_wait(b):
            pltpu.make_async_copy(
                table_hbm.at[idx_v.at[0]], bufs.at[b], gsems[b]
            ).wait()

        def out_copy(j, b):
            pltpu.sync_copy(bufs.at[b], out_hbm.at[pl.ds(base + j * CHUNK, CHUNK)])

        for b in range(K):
            gather_fire(b, b)

        @pl.loop(0, n_groups)
        def _step(g):
            j0 = g * K
            for b in range(K):
                gather_wait(b)
                out_copy(j0 + b, b)
                gather_fire(j0 + b + K, b)

        for j in range(n_groups * K, n_chunks - K):
            b = j % K
            gather_wait(b)
            out_copy(j, b)
            gather_fire(j + K, b)

        for j in range(n_chunks - K, n_chunks):
            b = j % K
            gather_wait(b)
            out_copy(j, b)

    return lookup


def kernel(token_ids, table):
    B0, S = token_ids.shape
    B = B0 * S
    idx = token_ids.astype(jnp.int32).T.reshape(NW, B // (NW * CHUNK), CHUNK)
    out = _make_lookup(B)(idx, table)
    return out.reshape(S, B0, D_MODEL).transpose(1, 0, 2)


# speedup vs baseline: 1.0061x; 1.0061x over previous
import functools

import jax
import jax.numpy as jnp
from jax import lax
from jax.experimental import pallas as pl
from jax.experimental.pallas import tpu as pltpu
from jax.experimental.pallas import tpu_sc as plsc

D_MODEL = 128
NC, NS = 2, 16
NW = NC * NS
CHUNK = 128
K = 7


def _make_lookup(B):
    assert B % (NW * CHUNK) == 0
    n_chunks = B // (NW * CHUNK)
    n_groups, rem = divmod(n_chunks - K, K)
    mesh = plsc.VectorSubcoreMesh(core_axis_name="c", subcore_axis_name="s")

    @functools.partial(
        pl.kernel,
        out_type=jax.ShapeDtypeStruct((B, D_MODEL), jnp.float32),
        mesh=mesh,
        scratch_types=[
            pltpu.VMEM((n_chunks, CHUNK), jnp.int32),
            pltpu.VMEM((K, CHUNK, D_MODEL), jnp.float32),
        ]
        + [pltpu.SemaphoreType.DMA] * K,
    )
    def lookup(idx_hbm, table_hbm, out_hbm, idx_v, bufs, *gsems):
        wid = lax.axis_index("s") * NC + lax.axis_index("c")
        base = wid * (n_chunks * CHUNK)
        pltpu.sync_copy(idx_hbm.at[wid], idx_v)

        def gather_fire(j, b):
            pltpu.async_copy(table_hbm.at[idx_v.at[j]], bufs.at[b], gsems[b])

        def gather_wait(b):
            pltpu.make_async_copy(
                table_hbm.at[idx_v.at[0]], bufs.at[b], gsems[b]
            ).wait()

        def out_copy(j, b):
            pltpu.sync_copy(bufs.at[b], out_hbm.at[pl.ds(base + j * CHUNK, CHUNK)])

        for b in range(K):
            gather_fire(b, b)

        @pl.loop(0, n_groups)
        def _step(g):
            j0 = g * K
            for b in range(K):
                gather_wait(b)
                out_copy(j0 + b, b)
                gather_fire(j0 + b + K, b)

        for j in range(n_groups * K, n_chunks - K):
            b = j % K
            gather_wait(b)
            out_copy(j, b)
            gather_fire(j + K, b)

        for j in range(n_chunks - K, n_chunks):
            b = j % K
            gather_wait(b)
            out_copy(j, b)

    return lookup


def kernel(token_ids, table):
    B0, S = token_ids.shape
    B = B0 * S
    idx = token_ids.astype(jnp.int32).T.reshape(NW, B // (NW * CHUNK), CHUNK)
    out = _make_lookup(B)(idx, table)
    return out.reshape(S, B0, D_MODEL).transpose(1, 0, 2)


# speedup vs baseline: 1.0441x; 1.0377x over previous
import functools

import jax
import jax.numpy as jnp
from jax import lax
from jax.experimental import pallas as pl
from jax.experimental.pallas import tpu as pltpu
from jax.experimental.pallas import tpu_sc as plsc

D_MODEL = 128
NC, NS = 2, 16
NW = NC * NS
CHUNK = 128
K = 7


def _make_lookup(S, B0):
    assert B0 == NW * CHUNK
    n_chunks = S
    n_groups, _ = divmod(n_chunks - K, K)
    mesh = plsc.VectorSubcoreMesh(core_axis_name="c", subcore_axis_name="s")

    @functools.partial(
        pl.kernel,
        out_type=jax.ShapeDtypeStruct((S * B0, D_MODEL), jnp.float32),
        mesh=mesh,
        scratch_types=[
            pltpu.VMEM((n_chunks, CHUNK), jnp.int32),
            pltpu.VMEM((K, CHUNK, D_MODEL), jnp.float32),
        ]
        + [pltpu.SemaphoreType.DMA] * K,
    )
    def lookup(idx_hbm, table_hbm, out_hbm, idx_v, bufs, *gsems):
        wid = lax.axis_index("s") * NC + lax.axis_index("c")
        pltpu.sync_copy(idx_hbm.at[:, pl.ds(wid * CHUNK, CHUNK)], idx_v)

        def gather_fire(j, b):
            pltpu.async_copy(table_hbm.at[idx_v.at[j]], bufs.at[b], gsems[b])

        def gather_wait(b):
            pltpu.make_async_copy(
                table_hbm.at[idx_v.at[0]], bufs.at[b], gsems[b]
            ).wait()

        def out_copy(j, b):
            pltpu.sync_copy(
                bufs.at[b], out_hbm.at[pl.ds(j * B0 + wid * CHUNK, CHUNK)]
            )

        for b in range(K):
            gather_fire(b, b)

        @pl.loop(0, n_groups)
        def _step(g):
            j0 = g * K
            for b in range(K):
                gather_wait(b)
                out_copy(j0 + b, b)
                gather_fire(j0 + b + K, b)

        for j in range(n_groups * K, n_chunks - K):
            b = j % K
            gather_wait(b)
            out_copy(j, b)
            gather_fire(j + K, b)

        for j in range(n_chunks - K, n_chunks):
            b = j % K
            gather_wait(b)
            out_copy(j, b)

    return lookup


def kernel(token_ids, table):
    B0, S = token_ids.shape
    idx = token_ids.astype(jnp.int32).T
    out = _make_lookup(S, B0)(idx, table)
    return out.reshape(S, B0, D_MODEL).transpose(1, 0, 2)
